# Initial kernel scaffold; baseline (speedup 1.0000x reference)
import jax
import jax.numpy as jnp
from jax.experimental import pallas as pl


def kernel(x, token_table, pos_table):
    raise NotImplementedError("write your pallas kernel here")


# speedup vs baseline: 3.0996x; 3.0996x over previous
import functools

import jax
import jax.numpy as jnp
from jax import lax
from jax.experimental import pallas as pl
from jax.experimental.pallas import tpu as pltpu
from jax.experimental.pallas import tpu_sc as plsc

VOCAB = 100000
MAXLEN = 200
EMBED = 64
BATCH = 4096

NC = 2
NS = 16
NW = NC * NS
ROWS_PER_W = BATCH // NW
CHUNK0 = 104
CHUNK1 = MAXLEN - CHUNK0


def _body(x_hbm, tok_hbm, pos_hbm, out_hbm, idx_v, buf_v, pos_v, sem):
    wid = lax.axis_index("s") * NC + lax.axis_index("c")
    base = wid * ROWS_PER_W

    pltpu.sync_copy(pos_hbm, pos_v)

    def row_fn(r, carry):
        row = base + r
        pltpu.sync_copy(x_hbm.at[row], idx_v)
        cp0 = pltpu.async_copy(
            tok_hbm.at[idx_v.at[pl.ds(0, CHUNK0)]],
            buf_v.at[pl.ds(0, CHUNK0)], sem)
        cp1 = pltpu.async_copy(
            tok_hbm.at[idx_v.at[pl.ds(CHUNK0, CHUNK1)]],
            buf_v.at[pl.ds(CHUNK0, CHUNK1)], sem)
        cp0.wait()
        cp1.wait()

        def add_fn(m, c):
            for d in range(EMBED // 16):
                sl = (m, pl.ds(d * 16, 16))
                buf_v[sl] = buf_v[sl] + pos_v[sl]
            return c

        lax.fori_loop(0, MAXLEN, add_fn, 0)
        pltpu.sync_copy(buf_v, out_hbm.at[row])
        return carry

    lax.fori_loop(0, ROWS_PER_W, row_fn, 0)


@jax.jit
def kernel(x, token_table, pos_table):
    mesh = plsc.VectorSubcoreMesh(
        core_axis_name="c", subcore_axis_name="s",
        num_cores=NC, num_subcores=NS)
    f = pl.kernel(
        _body,
        out_type=jax.ShapeDtypeStruct((BATCH, MAXLEN, EMBED), jnp.float32),
        mesh=mesh,
        scratch_types=[
            pltpu.VMEM((MAXLEN,), jnp.int32),
            pltpu.VMEM((MAXLEN, EMBED), jnp.float32),
            pltpu.VMEM((MAXLEN, EMBED), jnp.float32),
            pltpu.SemaphoreType.DMA,
        ],
        compiler_params=pltpu.CompilerParams(use_tc_tiling_on_sc=False),
    )
    return f(x.astype(jnp.int32), token_table, pos_table)


# speedup vs baseline: 4.1422x; 1.3364x over previous
import functools

import jax
import jax.numpy as jnp
from jax import lax
from jax.experimental import pallas as pl
from jax.experimental.pallas import tpu as pltpu
from jax.experimental.pallas import tpu_sc as plsc

VOCAB = 100000
MAXLEN = 200
EMBED = 64
BATCH = 4096

NC = 2
NS = 16
NW = NC * NS
ROWS_PER_W = BATCH // NW
CHUNK0 = 104
CHUNK1 = MAXLEN - CHUNK0
NBUF = 4


def _gather_descs(tok_hbm, idx_all, buf, sem, lr):
    return (
        pltpu.make_async_copy(
            tok_hbm.at[idx_all.at[lr, pl.ds(0, CHUNK0)]],
            buf.at[pl.ds(0, CHUNK0)], sem),
        pltpu.make_async_copy(
            tok_hbm.at[idx_all.at[lr, pl.ds(CHUNK0, CHUNK1)]],
            buf.at[pl.ds(CHUNK0, CHUNK1)], sem),
    )


def _body(x_hbm, tok_hbm, pos_hbm, out_hbm,
          idx_all, pos_v, bufs, sems_g, sems_o):
    wid = lax.axis_index("s") * NC + lax.axis_index("c")
    base = wid * ROWS_PER_W

    pltpu.sync_copy(pos_hbm, pos_v)
    pltpu.sync_copy(x_hbm.at[pl.ds(base, ROWS_PER_W)], idx_all)

    def start_gather(lr, b):
        for d in _gather_descs(tok_hbm, idx_all, bufs[b], sems_g[b], lr):
            d.start()

    def wait_gather(lr, b):
        for d in _gather_descs(tok_hbm, idx_all, bufs[b], sems_g[b], lr):
            d.wait()

    start_gather(0, 0)
    start_gather(1, 1)

    def step(lr, b):
        wait_gather(lr, b)

        @plsc.parallel_loop(0, MAXLEN, unroll=2)
        def _add(m):
            for d in range(EMBED // 16):
                sl = (m, pl.ds(d * 16, 16))
                plsc.addupdate(bufs[b].at[sl], pos_v[sl])

        pltpu.make_async_copy(bufs[b], out_hbm.at[base + lr], sems_o[b]).start()

        b2 = (b + 2) % NBUF

        @pl.when(lr + 2 < ROWS_PER_W)
        def _():
            @pl.when(lr >= 2)
            def _():
                pltpu.make_async_copy(
                    bufs[b2], out_hbm.at[base + lr - 2], sems_o[b2]).wait()
            start_gather(lr + 2, b2)

    def group(g, carry):
        for b in range(NBUF):
            step(g * NBUF + b, b)
        return carry

    lax.fori_loop(0, ROWS_PER_W // NBUF, group, 0)

    for lr in (ROWS_PER_W - 2, ROWS_PER_W - 1):
        b = lr % NBUF
        pltpu.make_async_copy(bufs[b], out_hbm.at[base + lr], sems_o[b]).wait()


@jax.jit
def kernel(x, token_table, pos_table):
    mesh = plsc.VectorSubcoreMesh(
        core_axis_name="c", subcore_axis_name="s",
        num_cores=NC, num_subcores=NS)

    def body(x_hbm, tok_hbm, pos_hbm, out_hbm, idx_all, pos_v,
             b0, b1, b2, b3, g0, g1, g2, g3, o0, o1, o2, o3):
        _body(x_hbm, tok_hbm, pos_hbm, out_hbm, idx_all, pos_v,
              (b0, b1, b2, b3), (g0, g1, g2, g3), (o0, o1, o2, o3))

    f = pl.kernel(
        body,
        out_type=jax.ShapeDtypeStruct((BATCH, MAXLEN, EMBED), jnp.float32),
        mesh=mesh,
        scratch_types=(
            [pltpu.VMEM((ROWS_PER_W, MAXLEN), jnp.int32),
             pltpu.VMEM((MAXLEN, EMBED), jnp.float32)]
            + [pltpu.VMEM((MAXLEN, EMBED), jnp.float32)] * NBUF
            + [pltpu.SemaphoreType.DMA] * (2 * NBUF)
        ),
        compiler_params=pltpu.CompilerParams(use_tc_tiling_on_sc=False),
    )
    return f(x.astype(jnp.int32), token_table, pos_table)
